# Initial kernel scaffold; baseline (speedup 1.0000x reference)
import jax
import jax.numpy as jnp
from jax.experimental import pallas as pl


def kernel(visual_features, texts, embed_table, proj_W, proj_b, image_token_id):
    raise NotImplementedError("write your pallas kernel here")


# speedup vs baseline: 4.1892x; 4.1892x over previous
import functools

import jax
import jax.numpy as jnp
from jax import lax
from jax.experimental import pallas as pl
from jax.experimental.pallas import tpu as pltpu
from jax.experimental.pallas import tpu_sc as plsc

_NC = 2
_NS = 16
_NW = _NC * _NS


def _proj_body(x_ref, w_ref, b_ref, o_ref):
    o_ref[...] = (
        jnp.dot(x_ref[...], w_ref[...], preferred_element_type=jnp.float32)
        + b_ref[...]
    )


def _project(x, w, b):
    m, d = x.shape
    blk_m = 256
    return pl.pallas_call(
        _proj_body,
        grid=(m // blk_m,),
        in_specs=[
            pl.BlockSpec((blk_m, d), lambda i: (i, 0)),
            pl.BlockSpec((d, d), lambda i: (0, 0)),
            pl.BlockSpec((1, d), lambda i: (0, 0)),
        ],
        out_specs=pl.BlockSpec((blk_m, d), lambda i: (i, 0)),
        out_shape=jax.ShapeDtypeStruct((m, d), jnp.float32),
    )(x, w, b.reshape(1, d))


def _mask_body(texts_ref, img_ref, o_ref, *, L, P):
    row = texts_ref[...]
    img = img_ref[0, 0]
    s = row.shape[-1]
    io = lax.broadcasted_iota(jnp.int32, (1, 1, s), 2)
    pos = jnp.max(jnp.where(row == img, io, -1))
    q = lax.broadcasted_iota(jnp.int32, (1, 1, L, L), 2)
    k = lax.broadcasted_iota(jnp.int32, (1, 1, L, L), 3)
    vis_q = (q >= pos) & (q < pos + P)
    vis_k = (k >= pos) & (k < pos + P)
    m = (q >= k) | (vis_q & vis_k)
    o_ref[...] = m.astype(jnp.float32)


def _mask(texts, img11, L, P):
    b, s = texts.shape
    return pl.pallas_call(
        functools.partial(_mask_body, L=L, P=P),
        grid=(b,),
        in_specs=[
            pl.BlockSpec((1, 1, s), lambda i: (i, 0, 0)),
            pl.BlockSpec((1, 1), lambda i: (0, 0)),
        ],
        out_specs=pl.BlockSpec((1, 1, L, L), lambda i: (i, 0, 0, 0)),
        out_shape=jax.ShapeDtypeStruct((b, 1, L, L), jnp.float32),
    )(texts.reshape(b, 1, s), img11)


def _assemble_body(texts_hbm, img_hbm, embed_hbm, proj_hbm, out_hbm,
                   texts_v, img_v, tok_v, dst_v, rows_v, sem,
                   *, S, P, D, L, V):
    wid = lax.axis_index("s") * _NC + lax.axis_index("c")
    b = wid // 4
    q = wid % 4

    pltpu.sync_copy(texts_hbm.at[pl.ds(b * S, S)], texts_v.at[pl.ds(0, S)])
    pltpu.sync_copy(img_hbm, img_v)
    img = img_v[...]

    io16 = lax.broadcasted_iota(jnp.int32, (16,), 0)
    zero16 = jnp.zeros((16,), jnp.int32)

    def _pos_step(i, acc):
        t = texts_v[pl.ds(i * 16, 16)]
        return acc + jnp.where(t == img, io16 + i * 16, 0)

    acc = lax.fori_loop(0, S // 16, _pos_step, jnp.zeros((16,), jnp.int32))
    pos = acc[0]
    for i in range(1, 16):
        pos = pos + acc[i]

    u0 = q * 128
    for j in range(8):
        off = u0 + j * 16
        u = off + io16
        t0 = texts_v[pl.ds(off, 16)]
        t1 = texts_v[pl.ds(off + 1, 16)]
        is_pre = u < pos
        tok = jnp.clip(jnp.where(is_pre, t0, t1), 0, V - 1)
        t_out = jnp.where(is_pre, u, u + P)
        dest = b * L + jnp.where(u == S - 1, pos + 192, t_out)
        c = j // 4
        o = (j % 4) * 16
        tok_v[c, pl.ds(o, 16)] = tok
        dst_v[c, pl.ds(o, 16)] = dest

    v0 = b * L + pos + q * 64
    for j in range(4):
        dst_v[2, pl.ds(j * 16, 16)] = v0 + j * 16 + io16

    for c in range(2):
        pltpu.async_copy(embed_hbm.at[tok_v.at[c]], rows_v, sem).wait()
        pltpu.async_copy(rows_v, out_hbm.at[dst_v.at[c]], sem).wait()

    src0 = b * P + q * 64
    pltpu.sync_copy(proj_hbm.at[pl.ds(src0, 64)], rows_v)
    pltpu.async_copy(rows_v, out_hbm.at[dst_v.at[2]], sem).wait()


def _assemble(texts, img16, embed_table, projected, S, P, D, L):
    b = texts.shape[0]
    body = functools.partial(_assemble_body, S=S, P=P, D=D, L=L,
                             V=embed_table.shape[0])
    k = pl.kernel(
        body,
        out_type=jax.ShapeDtypeStruct((b * L, D), jnp.float32),
        mesh=plsc.VectorSubcoreMesh(core_axis_name="c", subcore_axis_name="s"),
        scratch_types=[
            pltpu.VMEM((S + 16,), jnp.int32),
            pltpu.VMEM((16,), jnp.int32),
            pltpu.VMEM((2, 64), jnp.int32),
            pltpu.VMEM((3, 64), jnp.int32),
            pltpu.VMEM((64, D), jnp.float32),
            pltpu.SemaphoreType.DMA,
        ],
    )
    return k(texts.reshape(b * S), img16, embed_table, projected)


def kernel(visual_features, texts, embed_table, proj_W, proj_b, image_token_id):
    b, s = texts.shape
    p = visual_features.shape[1]
    d = visual_features.shape[2]
    L = s - 1 + p

    projected = _project(visual_features.reshape(b * p, d), proj_W, proj_b)

    img16 = jnp.full((16,), image_token_id, dtype=jnp.int32)
    flat = _assemble(texts, img16, embed_table, projected, s, p, d, L)
    emb = flat.reshape(b, L, d)

    img11 = jnp.asarray(image_token_id, jnp.int32).reshape(1, 1)
    mask = _mask(texts, img11, L, p)
    return emb, mask
